# baseline (speedup 1.0000x reference)
import functools

import jax
import jax.numpy as jnp
from jax import lax
from jax.experimental import pallas as pl
from jax.experimental.pallas import tpu as pltpu
from jax.experimental.pallas import tpu_sc as plsc

N_NODES = 50000
N_EDGES = 800000
RBF_DIM = 128
DIM = 64
HALF = DIM // 2


_BR = 3200
_GRID = N_EDGES // _BR


def _mlp_body(rbf_ref, ef_ref, w1_ref, b1_ref, w2_ref, b2_ref,
              h2_ref, ef2_ref):
    x = jnp.dot(rbf_ref[...], w1_ref[...],
                preferred_element_type=jnp.float32) + b1_ref[...]
    bx = 0.5 * x
    sp = 2.0 * (jnp.maximum(bx, 0.0) + jnp.log1p(jnp.exp(-jnp.abs(bx))))
    xs = jnp.where(bx > 14.0, x, sp)
    y = jnp.dot(xs, w2_ref[...],
                preferred_element_type=jnp.float32) + b2_ref[...]
    q = _BR // 4
    h2_ref[0] = jnp.concatenate(
        [y[k * q:(k + 1) * q, :HALF] for k in range(4)], axis=1)
    h2_ref[1] = jnp.concatenate(
        [y[k * q:(k + 1) * q, HALF:] for k in range(4)], axis=1)
    ef = ef_ref[...]
    ef2_ref[0] = jnp.concatenate(
        [ef[k * q:(k + 1) * q, :HALF] for k in range(4)], axis=1)
    ef2_ref[1] = jnp.concatenate(
        [ef[k * q:(k + 1) * q, HALF:] for k in range(4)], axis=1)


def _run_mlp(rbf, edge_f, W1, b1, W2, b2):
    h2, ef2 = pl.pallas_call(
        _mlp_body,
        grid=(_GRID,),
        in_specs=[
            pl.BlockSpec((_BR, RBF_DIM), lambda i: (i, 0)),
            pl.BlockSpec((_BR, DIM), lambda i: (i, 0)),
            pl.BlockSpec((RBF_DIM, DIM), lambda i: (0, 0)),
            pl.BlockSpec((1, DIM), lambda i: (0, 0)),
            pl.BlockSpec((DIM, DIM), lambda i: (0, 0)),
            pl.BlockSpec((1, DIM), lambda i: (0, 0)),
        ],
        out_specs=[
            pl.BlockSpec((2, _BR // 4, 128), lambda i: (0, i, 0)),
            pl.BlockSpec((2, _BR // 4, 128), lambda i: (0, i, 0)),
        ],
        out_shape=[
            jax.ShapeDtypeStruct((2, N_EDGES // 4, 128), jnp.float32),
            jax.ShapeDtypeStruct((2, N_EDGES // 4, 128), jnp.float32),
        ],
    )(rbf, edge_f, W1, b1.reshape(1, DIM), W2, b2.reshape(1, DIM))
    return h2, ef2


_NSUB = 16
_EPS = N_EDGES // _NSUB
_B = 80
_NB = _EPS // _B
_NSLOT = 3
_ACC_ROWS = 50176
_ZROWS = _ACC_ROWS // _NSUB
_ZCH = 112
_NZ = _ZROWS // _ZCH
_OCH = 224
_NOCH = _ZROWS // _OCH


def _sc_body(nn_hbm, h_hbm, ef_hbm, src_hbm, dst_hbm, out_hbm,
             acc, gidx, sdid, rows, hbuf, efbuf, sems):
    cid = lax.axis_index("c")
    sid = lax.axis_index("s")
    sem_in = sems[0:3]
    sem_g = sems[3:6]
    sem_s = sems[6:9]

    def _zb(i, _):
        r = i // 2
        c = (i % 2) * 16
        rows[r, pl.ds(c, 16)] = jnp.zeros((16,), jnp.float32)
        return 0
    lax.fori_loop(0, 2 * _ZCH, _zb, 0)

    def _za(j, _):
        pltpu.sync_copy(rows.at[pl.ds(0, _ZCH)],
                        acc.at[pl.ds(sid * _ZROWS + j * _ZCH, _ZCH)])
        return 0
    lax.fori_loop(0, _NZ, _za, 0)
    plsc.subcore_barrier()

    ebase = sid * _EPS
    hoff = cid * N_EDGES

    def _in_descs(b, s):
        base = ebase + b * _B
        hrow = (hoff + base) // 4
        return (
            (src_hbm.at[pl.ds(hoff + base, _B)],
             gidx.at[pl.ds(s * _B, _B)], sem_in[s]),
            (dst_hbm.at[pl.ds(sid * _NB + b, 1)],
             sdid.at[pl.ds(s, 1)], sem_in[s]),
            (h_hbm.at[pl.ds(hrow, _B // 4)],
             hbuf.at[pl.ds(s * (_B // 4), _B // 4)], sem_in[s]),
            (ef_hbm.at[pl.ds(hrow, _B // 4)],
             efbuf.at[pl.ds(s * (_B // 4), _B // 4)], sem_in[s]),
        )

    def _gather_desc(s):
        return (nn_hbm.at[gidx.at[pl.ds(s * _B, _B)]],
                rows.at[pl.ds(s * _B, _B)], sem_g[s])

    def _scatter_desc(s):
        return (rows.at[pl.ds(s * _B, _B)], acc.at[sdid.at[s]], sem_s[s])

    def _stage_a(b, s):
        @pl.when(jnp.logical_and(b >= _NSLOT, b < _NB))
        def _():
            sr, dsr, sem = _scatter_desc(s)
            pltpu.make_async_copy(sr, dsr, sem).wait()

        @pl.when(b < _NB)
        def _():
            for sr, dsr, sem in _in_descs(b, s):
                pltpu.async_copy(sr, dsr, sem)

    def _stage_b(b, s):
        @pl.when(jnp.logical_and(b >= 0, b < _NB))
        def _():
            for sr, dsr, sem in _in_descs(b, s):
                pltpu.make_async_copy(sr, dsr, sem).wait()
            sr, dsr, sem = _gather_desc(s)
            pltpu.async_copy(sr, dsr, sem)

    def _stage_c(b, s):
        @pl.when(jnp.logical_and(b >= 0, b < _NB))
        def _():
            sr, dsr, sem = _gather_desc(s)
            pltpu.make_async_copy(sr, dsr, sem).wait()

            def _mac(r, _):
                hr = s * (_B // 4) + r
                for u in range(8):
                    ri = s * _B + r * 4 + u // 2
                    sl = pl.ds((u % 2) * 16, 16)
                    hl = pl.ds(u * 16, 16)
                    rows[ri, sl] = (rows[ri, sl] * hbuf[hr, hl]
                                    + efbuf[hr, hl])
                return 0
            lax.fori_loop(0, _B // 4, _mac, 0)
            sr, dsr, sem = _scatter_desc(s)
            pltpu.async_copy(sr, dsr, sem, add=True)

    def _step(t, _):
        for s in range(_NSLOT):
            b = t * _NSLOT + s
            _stage_a(b, s)
            _stage_b(b - 1, (s + 2) % 3)
            _stage_c(b - 2, (s + 1) % 3)
        return 0
    lax.fori_loop(0, (_NB + 2 + _NSLOT - 1) // _NSLOT, _step, 0)

    for s in range(_NSLOT):
        sr, dsr, sem = _scatter_desc(s)
        pltpu.make_async_copy(sr, dsr, sem).wait()
    plsc.subcore_barrier()

    def _och(ch, _):
        arow = sid * _ZROWS + ch * _OCH
        pltpu.sync_copy(acc.at[pl.ds(arow, _OCH)], rows.at[pl.ds(0, _OCH)])

        def _rp(i, _):
            hbuf[i // 8, pl.ds((i % 8) * 16, 16)] = \
                rows[i // 2, pl.ds((i % 2) * 16, 16)]
            return 0
        lax.fori_loop(0, 2 * _OCH, _rp, 0)
        orow = (sid * _ZROWS + ch * _OCH) // 4
        pltpu.sync_copy(hbuf.at[pl.ds(0, _OCH // 4)],
                        out_hbm.at[cid, pl.ds(orow, _OCH // 4)])
        return 0
    lax.fori_loop(0, _NOCH, _och, 0)


@functools.partial(
    pl.kernel,
    out_type=jax.ShapeDtypeStruct((2, _ACC_ROWS // 4, 128), jnp.float32),
    mesh=plsc.VectorSubcoreMesh(core_axis_name="c", subcore_axis_name="s"),
    compiler_params=pltpu.CompilerParams(use_tc_tiling_on_sc=False),
    scratch_types=[
        pltpu.VMEM_SHARED((_ACC_ROWS, HALF), jnp.float32),
        pltpu.VMEM((_NSLOT * _B,), jnp.int32),
        pltpu.VMEM((_NSLOT, _B), jnp.int32),
        pltpu.VMEM((_NSLOT * _B, HALF), jnp.float32),
        pltpu.VMEM((_NSLOT * (_B // 4), 128), jnp.float32),
        pltpu.VMEM((_NSLOT * (_B // 4), 128), jnp.float32),
    ] + [pltpu.SemaphoreType.DMA] * 9,
)
def _sc_kernel(nn_hbm, h_hbm, ef_hbm, src_hbm, dst_hbm, out_hbm,
               acc, gidx, sdid, rows, hbuf, efbuf, *sems):
    _sc_body(nn_hbm, h_hbm, ef_hbm, src_hbm, dst_hbm, out_hbm,
             acc, gidx, sdid, rows, hbuf, efbuf, list(sems))


def kernel(new_node, rbf, edge_f, edge_index, W1, b1, W2, b2):
    h2, ef2 = _run_mlp(rbf, edge_f, W1, b1, W2, b2)
    hf = h2.reshape(N_EDGES // 2, 128)
    eff = ef2.reshape(N_EDGES // 2, 128)
    nn2 = jnp.concatenate([new_node[:, :HALF], new_node[:, HALF:]], axis=0)

    def _perm(a):
        return a.reshape(N_EDGES // _BR, 4, _BR // 4).transpose(0, 2, 1) \
                .reshape(N_EDGES)

    src = _perm(edge_index[0].astype(jnp.int32))
    srcx = jnp.concatenate([src, src + N_NODES])
    dst = _perm(edge_index[1].astype(jnp.int32)).reshape(N_EDGES // _B, _B)
    out2 = _sc_kernel(nn2, hf, eff, srcx, dst)
    o = out2.reshape(2, _ACC_ROWS, HALF)[:, :N_NODES]
    return o.transpose(1, 0, 2).reshape(N_NODES, DIM)


# speedup vs baseline: 1.0783x; 1.0783x over previous
import functools

import jax
import jax.numpy as jnp
import numpy as np
from jax import lax
from jax.experimental import pallas as pl
from jax.experimental.pallas import tpu as pltpu
from jax.experimental.pallas import tpu_sc as plsc

N_NODES = 50000
N_EDGES = 800000
RBF_DIM = 128
DIM = 64
HALF = DIM // 2


_BR = 3200
_GRID = N_EDGES // _BR


def _mlp_body(rbf_ref, ef_ref, w1_ref, b1_ref, w2_ref, b2_ref,
              h2_ref, ef2_ref):
    x = jnp.dot(rbf_ref[...], w1_ref[...],
                preferred_element_type=jnp.float32) + b1_ref[...]
    bx = 0.5 * x
    sp = 2.0 * (jnp.maximum(bx, 0.0) + jnp.log1p(jnp.exp(-jnp.abs(bx))))
    xs = jnp.where(bx > 14.0, x, sp)
    y = jnp.dot(xs, w2_ref[...],
                preferred_element_type=jnp.float32) + b2_ref[...]
    q = _BR // 4
    h2_ref[0] = jnp.concatenate(
        [y[k * q:(k + 1) * q, :HALF] for k in range(4)], axis=1)
    h2_ref[1] = jnp.concatenate(
        [y[k * q:(k + 1) * q, HALF:] for k in range(4)], axis=1)
    ef = ef_ref[...]
    ef2_ref[0] = jnp.concatenate(
        [ef[k * q:(k + 1) * q, :HALF] for k in range(4)], axis=1)
    ef2_ref[1] = jnp.concatenate(
        [ef[k * q:(k + 1) * q, HALF:] for k in range(4)], axis=1)


def _run_mlp(rbf, edge_f, W1, b1, W2, b2):
    h2, ef2 = pl.pallas_call(
        _mlp_body,
        grid=(_GRID,),
        in_specs=[
            pl.BlockSpec((_BR, RBF_DIM), lambda i: (i, 0)),
            pl.BlockSpec((_BR, DIM), lambda i: (i, 0)),
            pl.BlockSpec((RBF_DIM, DIM), lambda i: (0, 0)),
            pl.BlockSpec((1, DIM), lambda i: (0, 0)),
            pl.BlockSpec((DIM, DIM), lambda i: (0, 0)),
            pl.BlockSpec((1, DIM), lambda i: (0, 0)),
        ],
        out_specs=[
            pl.BlockSpec((2, _BR // 4, 128), lambda i: (0, i, 0)),
            pl.BlockSpec((2, _BR // 4, 128), lambda i: (0, i, 0)),
        ],
        out_shape=[
            jax.ShapeDtypeStruct((2, N_EDGES // 4, 128), jnp.float32),
            jax.ShapeDtypeStruct((2, N_EDGES // 4, 128), jnp.float32),
        ],
    )(rbf, edge_f, W1, b1.reshape(1, DIM), W2, b2.reshape(1, DIM))
    return h2, ef2


_NSUB = 16
_EPS = N_EDGES // _NSUB
_B = 80
_NB = _EPS // _B
_NSLOT = 3
_ACC_ROWS = 50176
_ZROWS = _ACC_ROWS // _NSUB
_ZCH = 112
_NZ = _ZROWS // _ZCH
_OCH = 224
_NOCH = _ZROWS // _OCH


def _sc_body(nn_hbm, h_hbm, ef_hbm, src_hbm, dst_hbm, out_hbm,
             acc, gidx, sdid, rows, hbuf, efbuf, sems):
    cid = lax.axis_index("c")
    sid = lax.axis_index("s")
    sem_in = sems[0:3]
    sem_g = sems[3:6]
    sem_s = sems[6:9]

    def _zb(i, _):
        r = i // 2
        c = (i % 2) * 16
        rows[r, pl.ds(c, 16)] = jnp.zeros((16,), jnp.float32)
        return 0
    lax.fori_loop(0, 2 * _ZCH, _zb, 0)

    def _za(j, _):
        pltpu.sync_copy(rows.at[pl.ds(0, _ZCH)],
                        acc.at[pl.ds(sid * _ZROWS + j * _ZCH, _ZCH)])
        return 0
    lax.fori_loop(0, _NZ, _za, 0)
    plsc.subcore_barrier()

    ebase = sid * _EPS
    hoff = cid * N_EDGES

    def _in_descs(b, s):
        base = ebase + b * _B
        hrow = (hoff + base) // 4
        return (
            (src_hbm.at[pl.ds(base, _B)],
             gidx.at[pl.ds(s * _B, _B)], sem_in[s]),
            (dst_hbm.at[pl.ds(sid * _NB + b, 1)],
             sdid.at[pl.ds(s, 1)], sem_in[s]),
            (h_hbm.at[pl.ds(hrow, _B // 4)],
             hbuf.at[pl.ds(s * (_B // 4), _B // 4)], sem_in[s]),
            (ef_hbm.at[pl.ds(hrow, _B // 4)],
             efbuf.at[pl.ds(s * (_B // 4), _B // 4)], sem_in[s]),
        )

    def _gather_desc(s):
        return (nn_hbm.at[gidx.at[pl.ds(s * _B, _B)]],
                rows.at[pl.ds(s * _B, _B)], sem_g[s])

    def _scatter_desc(s):
        return (rows.at[pl.ds(s * _B, _B)], acc.at[sdid.at[s]], sem_s[s])

    def _stage_a(b, s):
        @pl.when(jnp.logical_and(b >= _NSLOT, b < _NB))
        def _():
            sr, dsr, sem = _scatter_desc(s)
            pltpu.make_async_copy(sr, dsr, sem).wait()

        @pl.when(b < _NB)
        def _():
            for sr, dsr, sem in _in_descs(b, s):
                pltpu.async_copy(sr, dsr, sem)

    goff = cid * N_NODES

    def _stage_b(b, s):
        @pl.when(jnp.logical_and(b >= 0, b < _NB))
        def _():
            for sr, dsr, sem in _in_descs(b, s):
                pltpu.make_async_copy(sr, dsr, sem).wait()
            for i in range(_B // 16):
                sl = pl.ds(s * _B + i * 16, 16)
                gidx[sl] = gidx[sl] + goff
            sr, dsr, sem = _gather_desc(s)
            pltpu.async_copy(sr, dsr, sem)

    def _stage_c(b, s):
        @pl.when(jnp.logical_and(b >= 0, b < _NB))
        def _():
            sr, dsr, sem = _gather_desc(s)
            pltpu.make_async_copy(sr, dsr, sem).wait()

            def _mac(r, _):
                hr = s * (_B // 4) + r
                for u in range(8):
                    ri = s * _B + r * 4 + u // 2
                    sl = pl.ds((u % 2) * 16, 16)
                    hl = pl.ds(u * 16, 16)
                    rows[ri, sl] = (rows[ri, sl] * hbuf[hr, hl]
                                    + efbuf[hr, hl])
                return 0
            lax.fori_loop(0, _B // 4, _mac, 0)
            sr, dsr, sem = _scatter_desc(s)
            pltpu.async_copy(sr, dsr, sem, add=True)

    def _step(t, _):
        for s in range(_NSLOT):
            b = t * _NSLOT + s
            _stage_a(b, s)
            _stage_b(b - 1, (s + 2) % 3)
            _stage_c(b - 2, (s + 1) % 3)
        return 0
    lax.fori_loop(0, (_NB + 2 + _NSLOT - 1) // _NSLOT, _step, 0)

    for s in range(_NSLOT):
        sr, dsr, sem = _scatter_desc(s)
        pltpu.make_async_copy(sr, dsr, sem).wait()
    plsc.subcore_barrier()

    def _och(ch, _):
        arow = sid * _ZROWS + ch * _OCH
        pltpu.sync_copy(acc.at[pl.ds(arow, _OCH)], rows.at[pl.ds(0, _OCH)])

        def _rp(i, _):
            hbuf[i // 8, pl.ds((i % 8) * 16, 16)] = \
                rows[i // 2, pl.ds((i % 2) * 16, 16)]
            return 0
        lax.fori_loop(0, 2 * _OCH, _rp, 0)
        orow = (sid * _ZROWS + ch * _OCH) // 4
        pltpu.sync_copy(hbuf.at[pl.ds(0, _OCH // 4)],
                        out_hbm.at[cid, pl.ds(orow, _OCH // 4)])
        return 0
    lax.fori_loop(0, _NOCH, _och, 0)


@functools.partial(
    pl.kernel,
    out_type=jax.ShapeDtypeStruct((2, _ACC_ROWS // 4, 128), jnp.float32),
    mesh=plsc.VectorSubcoreMesh(core_axis_name="c", subcore_axis_name="s"),
    compiler_params=pltpu.CompilerParams(use_tc_tiling_on_sc=False),
    scratch_types=[
        pltpu.VMEM_SHARED((_ACC_ROWS, HALF), jnp.float32),
        pltpu.VMEM((_NSLOT * _B,), jnp.int32),
        pltpu.VMEM((_NSLOT, _B), jnp.int32),
        pltpu.VMEM((_NSLOT * _B, HALF), jnp.float32),
        pltpu.VMEM((_NSLOT * (_B // 4), 128), jnp.float32),
        pltpu.VMEM((_NSLOT * (_B // 4), 128), jnp.float32),
    ] + [pltpu.SemaphoreType.DMA] * 9,
)
def _sc_kernel(nn_hbm, h_hbm, ef_hbm, src_hbm, dst_hbm, out_hbm,
               acc, gidx, sdid, rows, hbuf, efbuf, *sems):
    _sc_body(nn_hbm, h_hbm, ef_hbm, src_hbm, dst_hbm, out_hbm,
             acc, gidx, sdid, rows, hbuf, efbuf, list(sems))


def kernel(new_node, rbf, edge_f, edge_index, W1, b1, W2, b2):
    h2, ef2 = _run_mlp(rbf, edge_f, W1, b1, W2, b2)
    nn2 = jnp.concatenate([new_node[:, :HALF], new_node[:, HALF:]], axis=0)

    pidx = np.arange(N_EDGES).reshape(N_EDGES // _BR, 4, _BR // 4) \
             .transpose(0, 2, 1).reshape(N_EDGES)
    src = edge_index[0][pidx].astype(jnp.int32)
    dst = edge_index[1][pidx].astype(jnp.int32).reshape(N_EDGES // _B, _B)
    hf = h2.reshape(N_EDGES // 2, 128)
    eff = ef2.reshape(N_EDGES // 2, 128)
    out2 = _sc_kernel(nn2, hf, eff, src, dst)
    o = out2.reshape(2, _ACC_ROWS, HALF)[:, :N_NODES]
    return o.transpose(1, 0, 2).reshape(N_NODES, DIM)


# speedup vs baseline: 1.2708x; 1.1785x over previous
import functools

import jax
import jax.numpy as jnp
import numpy as np
from jax import lax
from jax.experimental import pallas as pl
from jax.experimental.pallas import tpu as pltpu
from jax.experimental.pallas import tpu_sc as plsc

N_NODES = 50000
N_EDGES = 800000
RBF_DIM = 128
DIM = 64
HALF = DIM // 2


_BR = 3200
_GRID = N_EDGES // _BR


def _mlp_body(rbf_ref, ef_ref, w1_ref, b1_ref, w2_ref, b2_ref,
              h2_ref, ef2_ref):
    x = jnp.dot(rbf_ref[...], w1_ref[...],
                preferred_element_type=jnp.float32) + b1_ref[...]
    bx = 0.5 * x
    sp = 2.0 * (jnp.maximum(bx, 0.0) + jnp.log1p(jnp.exp(-jnp.abs(bx))))
    xs = jnp.where(bx > 14.0, x, sp)
    y = jnp.dot(xs, w2_ref[...],
                preferred_element_type=jnp.float32) + b2_ref[...]
    q = _BR // 4
    h2_ref[0] = jnp.concatenate(
        [y[k * q:(k + 1) * q, :HALF] for k in range(4)], axis=1)
    h2_ref[1] = jnp.concatenate(
        [y[k * q:(k + 1) * q, HALF:] for k in range(4)], axis=1)
    ef = ef_ref[...].T
    ef2_ref[0] = jnp.concatenate(
        [ef[k * q:(k + 1) * q, :HALF] for k in range(4)], axis=1)
    ef2_ref[1] = jnp.concatenate(
        [ef[k * q:(k + 1) * q, HALF:] for k in range(4)], axis=1)


def _run_mlp(rbf, edge_f, W1, b1, W2, b2):
    h2, ef2 = pl.pallas_call(
        _mlp_body,
        grid=(_GRID,),
        in_specs=[
            pl.BlockSpec((_BR, RBF_DIM), lambda i: (i, 0)),
            pl.BlockSpec((DIM, _BR), lambda i: (0, i)),
            pl.BlockSpec((RBF_DIM, DIM), lambda i: (0, 0)),
            pl.BlockSpec((1, DIM), lambda i: (0, 0)),
            pl.BlockSpec((DIM, DIM), lambda i: (0, 0)),
            pl.BlockSpec((1, DIM), lambda i: (0, 0)),
        ],
        out_specs=[
            pl.BlockSpec((2, _BR // 4, 128), lambda i: (0, i, 0)),
            pl.BlockSpec((2, _BR // 4, 128), lambda i: (0, i, 0)),
        ],
        out_shape=[
            jax.ShapeDtypeStruct((2, N_EDGES // 4, 128), jnp.float32),
            jax.ShapeDtypeStruct((2, N_EDGES // 4, 128), jnp.float32),
        ],
    )(rbf, edge_f.T, W1, b1.reshape(1, DIM), W2, b2.reshape(1, DIM))
    return h2, ef2


_NSUB = 16
_EPS = N_EDGES // _NSUB
_B = 80
_NB = _EPS // _B
_NSLOT = 3
_ACC_ROWS = 50176
_ZROWS = _ACC_ROWS // _NSUB
_ZCH = 112
_NZ = _ZROWS // _ZCH
_OCH = 224
_NOCH = _ZROWS // _OCH


def _sc_body(nn_hbm, h_hbm, ef_hbm, src_hbm, dst_hbm, out_hbm,
             acc, gidx, sdid, rows, hbuf, efbuf, sems):
    cid = lax.axis_index("c")
    sid = lax.axis_index("s")
    sem_in = sems[0:3]
    sem_g = sems[3:6]
    sem_s = sems[6:9]

    def _zb(i, _):
        r = i // 2
        c = (i % 2) * 16
        rows[r, pl.ds(c, 16)] = jnp.zeros((16,), jnp.float32)
        return 0
    lax.fori_loop(0, 2 * _ZCH, _zb, 0)

    def _za(j, _):
        pltpu.sync_copy(rows.at[pl.ds(0, _ZCH)],
                        acc.at[pl.ds(sid * _ZROWS + j * _ZCH, _ZCH)])
        return 0
    lax.fori_loop(0, _NZ, _za, 0)
    plsc.subcore_barrier()

    ebase = sid * _EPS
    hoff = cid * N_EDGES

    def _in_descs(b, s):
        base = ebase + b * _B
        hrow = (hoff + base) // 4
        return (
            (src_hbm.at[pl.ds(base, _B)],
             gidx.at[pl.ds(s * _B, _B)], sem_in[s]),
            (dst_hbm.at[pl.ds(sid * _NB + b, 1)],
             sdid.at[pl.ds(s, 1)], sem_in[s]),
            (h_hbm.at[pl.ds(hrow, _B // 4)],
             hbuf.at[pl.ds(s * (_B // 4), _B // 4)], sem_in[s]),
            (ef_hbm.at[pl.ds(hrow, _B // 4)],
             efbuf.at[pl.ds(s * (_B // 4), _B // 4)], sem_in[s]),
        )

    def _gather_desc(s):
        return (nn_hbm.at[gidx.at[pl.ds(s * _B, _B)]],
                rows.at[pl.ds(s * _B, _B)], sem_g[s])

    def _scatter_desc(s):
        return (rows.at[pl.ds(s * _B, _B)], acc.at[sdid.at[s]], sem_s[s])

    def _stage_a(b, s):
        @pl.when(jnp.logical_and(b >= _NSLOT, b < _NB))
        def _():
            sr, dsr, sem = _scatter_desc(s)
            pltpu.make_async_copy(sr, dsr, sem).wait()

        @pl.when(b < _NB)
        def _():
            for sr, dsr, sem in _in_descs(b, s):
                pltpu.async_copy(sr, dsr, sem)

    goff = cid * N_NODES

    def _stage_b(b, s):
        @pl.when(jnp.logical_and(b >= 0, b < _NB))
        def _():
            for sr, dsr, sem in _in_descs(b, s):
                pltpu.make_async_copy(sr, dsr, sem).wait()
            for i in range(_B // 16):
                sl = pl.ds(s * _B + i * 16, 16)
                gidx[sl] = gidx[sl] + goff
            sr, dsr, sem = _gather_desc(s)
            pltpu.async_copy(sr, dsr, sem)

    def _stage_c(b, s):
        @pl.when(jnp.logical_and(b >= 0, b < _NB))
        def _():
            sr, dsr, sem = _gather_desc(s)
            pltpu.make_async_copy(sr, dsr, sem).wait()

            def _mac(r, _):
                hr = s * (_B // 4) + r
                for u in range(8):
                    ri = s * _B + r * 4 + u // 2
                    sl = pl.ds((u % 2) * 16, 16)
                    hl = pl.ds(u * 16, 16)
                    rows[ri, sl] = (rows[ri, sl] * hbuf[hr, hl]
                                    + efbuf[hr, hl])
                return 0
            lax.fori_loop(0, _B // 4, _mac, 0)
            sr, dsr, sem = _scatter_desc(s)
            pltpu.async_copy(sr, dsr, sem, add=True)

    def _step(t, _):
        for s in range(_NSLOT):
            b = t * _NSLOT + s
            _stage_a(b, s)
            _stage_b(b - 1, (s + 2) % 3)
            _stage_c(b - 2, (s + 1) % 3)
        return 0
    lax.fori_loop(0, (_NB + 2 + _NSLOT - 1) // _NSLOT, _step, 0)

    for s in range(_NSLOT):
        sr, dsr, sem = _scatter_desc(s)
        pltpu.make_async_copy(sr, dsr, sem).wait()
    plsc.subcore_barrier()

    def _och(ch, _):
        arow = sid * _ZROWS + ch * _OCH
        pltpu.sync_copy(acc.at[pl.ds(arow, _OCH)], rows.at[pl.ds(0, _OCH)])

        def _rp(i, _):
            hbuf[i // 8, pl.ds((i % 8) * 16, 16)] = \
                rows[i // 2, pl.ds((i % 2) * 16, 16)]
            return 0
        lax.fori_loop(0, 2 * _OCH, _rp, 0)
        orow = (sid * _ZROWS + ch * _OCH) // 4
        pltpu.sync_copy(hbuf.at[pl.ds(0, _OCH // 4)],
                        out_hbm.at[cid, pl.ds(orow, _OCH // 4)])
        return 0
    lax.fori_loop(0, _NOCH, _och, 0)


@functools.partial(
    pl.kernel,
    out_type=jax.ShapeDtypeStruct((2, _ACC_ROWS // 4, 128), jnp.float32),
    mesh=plsc.VectorSubcoreMesh(core_axis_name="c", subcore_axis_name="s"),
    compiler_params=pltpu.CompilerParams(use_tc_tiling_on_sc=False),
    scratch_types=[
        pltpu.VMEM_SHARED((_ACC_ROWS, HALF), jnp.float32),
        pltpu.VMEM((_NSLOT * _B,), jnp.int32),
        pltpu.VMEM((_NSLOT, _B), jnp.int32),
        pltpu.VMEM((_NSLOT * _B, HALF), jnp.float32),
        pltpu.VMEM((_NSLOT * (_B // 4), 128), jnp.float32),
        pltpu.VMEM((_NSLOT * (_B // 4), 128), jnp.float32),
    ] + [pltpu.SemaphoreType.DMA] * 9,
)
def _sc_kernel(nn_hbm, h_hbm, ef_hbm, src_hbm, dst_hbm, out_hbm,
               acc, gidx, sdid, rows, hbuf, efbuf, *sems):
    _sc_body(nn_hbm, h_hbm, ef_hbm, src_hbm, dst_hbm, out_hbm,
             acc, gidx, sdid, rows, hbuf, efbuf, list(sems))


def kernel(new_node, rbf, edge_f, edge_index, W1, b1, W2, b2):
    h2, ef2 = _run_mlp(rbf, edge_f, W1, b1, W2, b2)
    nn2 = jnp.concatenate([new_node[:, :HALF], new_node[:, HALF:]], axis=0)

    pidx = np.arange(N_EDGES).reshape(N_EDGES // _BR, 4, _BR // 4) \
             .transpose(0, 2, 1).reshape(N_EDGES)
    src = edge_index[0][pidx].astype(jnp.int32)
    dst = edge_index[1][pidx].astype(jnp.int32).reshape(N_EDGES // _B, _B)
    hf = h2.reshape(N_EDGES // 2, 128)
    eff = ef2.reshape(N_EDGES // 2, 128)
    out2 = _sc_kernel(nn2, hf, eff, src, dst)
    o = out2.reshape(2, _ACC_ROWS, HALF)[:, :N_NODES]
    return o.transpose(1, 0, 2).reshape(N_NODES, DIM)
